# baseline (speedup 1.0000x reference)
import functools

import jax
import jax.numpy as jnp
from jax import lax
from jax.experimental import pallas as pl
from jax.experimental.pallas import tpu as pltpu
from jax.experimental.pallas import tpu_sc as plsc

H = 128
NUM_FRAG = 50
N_CORES = 2
N_SUBCORES = 16
NW = N_CORES * N_SUBCORES
NBUF = 2
P = 4
BLOCK_G = 64


def _sc_embedding_sum(in_tab, out_tab, idx_in3, idx_out3):
    n_chunks, chunk = idx_in3.shape[1], idx_in3.shape[2]
    per_w = n_chunks * chunk
    B = NW * per_w
    mesh = plsc.VectorSubcoreMesh(core_axis_name="c", subcore_axis_name="s")

    @functools.partial(
        pl.kernel,
        mesh=mesh,
        out_type=jax.ShapeDtypeStruct((B, H), jnp.float32),
        scratch_types=[
            pltpu.VMEM_SHARED((512, H), jnp.float32),
            pltpu.VMEM_SHARED((512, H), jnp.float32),
            pltpu.VMEM((n_chunks, chunk), jnp.int32),
            pltpu.VMEM((n_chunks, chunk), jnp.int32),
        ]
        + [pltpu.VMEM((chunk, H), jnp.float32) for _ in range(3 * NBUF)]
        + [pltpu.SemaphoreType.DMA for _ in range(3 * NBUF)],
    )
    def k(in_tab_h, out_tab_h, ii_h, io_h, out_h, in_sp, out_sp, iiB, ioB,
          ri0, ri1, ro0, ro1, o0, o1, si0, si1, so0, so1, sw0, sw1):
        cid = lax.axis_index("c")
        sid = lax.axis_index("s")
        wid = sid * N_CORES + cid
        base = wid * per_w
        ri = [ri0, ri1]
        ro = [ro0, ro1]
        o = [o0, o1]
        si = [si0, si1]
        so = [so0, so1]
        sw = [sw0, sw1]

        @pl.when(sid == 0)
        def _():
            pltpu.sync_copy(in_tab_h, in_sp)
            pltpu.sync_copy(out_tab_h, out_sp)

        pltpu.sync_copy(ii_h.at[wid], iiB)
        pltpu.sync_copy(io_h.at[wid], ioB)
        plsc.subcore_barrier()

        def fill(b, ci):
            pltpu.async_copy(in_sp.at[iiB.at[ci]], ri[b], si[b])
            pltpu.async_copy(out_sp.at[ioB.at[ci]], ro[b], so[b])

        for b in range(NBUF):
            fill(b, b)

        def macro(m, carry):
            for b in range(NBUF):
                ci = NBUF * m + b
                off = base + ci * chunk
                pltpu.make_async_copy(in_sp.at[iiB.at[ci]], ri[b], si[b]).wait()
                pltpu.make_async_copy(out_sp.at[ioB.at[ci]], ro[b], so[b]).wait()

                @pl.when(m > 0)
                def _():
                    pltpu.make_async_copy(
                        o[b], out_h.at[pl.ds(base, chunk)], sw[b]).wait()

                @plsc.parallel_loop(0, chunk, unroll=2)
                def add_row(i):
                    for j in range(H // 16):
                        sl = pl.ds(j * 16, 16)
                        o[b][i, sl] = ri[b][i, sl] + ro[b][i, sl]
                pltpu.async_copy(o[b], out_h.at[pl.ds(off, chunk)], sw[b])

                @pl.when(ci + NBUF < n_chunks)
                def _():
                    fill(b, ci + NBUF)

            return carry

        lax.fori_loop(0, n_chunks // NBUF, macro, 0)

        for b in range(NBUF):
            pltpu.make_async_copy(
                o[b], out_h.at[pl.ds(base, chunk)], sw[b]).wait()

    return k(in_tab, out_tab, idx_in3, idx_out3)


def _tc_body(frag_ref, emb_ref, w_ref, b_ref, tok_ref, out_ref):
    x = frag_ref[...].reshape(BLOCK_G * NUM_FRAG, H)
    feat = lax.dot_general(
        x, w_ref[...], (((1,), (1,)), ((), ())),
        preferred_element_type=jnp.float32,
    )
    feat = feat + emb_ref[...] + b_ref[...]
    out_ref[:, 1:, :] = feat.reshape(BLOCK_G, NUM_FRAG, H)
    out_ref[:, 0:1, :] = jnp.broadcast_to(tok_ref[...][None, :, :],
                                          (BLOCK_G, 1, H))


def _tc_fuse_part(prev, frag_p, emb_p, W, b2, tok, n_graph, off_blocks):
    part_g = frag_p.shape[0]
    grid = part_g // BLOCK_G

    common = dict(
        grid=(grid,),
        out_specs=pl.BlockSpec((BLOCK_G, NUM_FRAG + 1, H),
                               lambda i: (i + off_blocks, 0, 0)),
        out_shape=jax.ShapeDtypeStruct((n_graph, NUM_FRAG + 1, H),
                                       jnp.float32),
        compiler_params=pltpu.CompilerParams(
            dimension_semantics=("arbitrary",)),
    )
    data_specs = [
        pl.BlockSpec((BLOCK_G, NUM_FRAG, H), lambda i: (i, 0, 0)),
        pl.BlockSpec((BLOCK_G * NUM_FRAG, H), lambda i: (i, 0)),
        pl.BlockSpec((H, H), lambda i: (0, 0)),
        pl.BlockSpec((1, H), lambda i: (0, 0)),
        pl.BlockSpec((1, H), lambda i: (0, 0)),
    ]
    if prev is None:
        return pl.pallas_call(
            _tc_body, in_specs=data_specs, **common,
        )(frag_p, emb_p, W, b2, tok)

    def body(prev_ref, frag_ref, emb_ref, w_ref, b_ref, tok_ref, out_ref):
        _tc_body(frag_ref, emb_ref, w_ref, b_ref, tok_ref, out_ref)

    return pl.pallas_call(
        body,
        in_specs=[pl.BlockSpec((1, NUM_FRAG + 1, H), lambda i: (0, 0, 0))]
        + data_specs,
        input_output_aliases={0: 0},
        **common,
    )(prev, frag_p, emb_p, W, b2, tok)


def kernel(frag_feature, in_degree, out_degree, W_feat, b_feat, in_tab,
           out_tab, graph_token):
    n_graph = frag_feature.shape[0]
    part_g = n_graph // P
    part_b = part_g * NUM_FRAG
    per_w = part_b // NW
    chunk = 128 if per_w % 128 == 0 else 80
    b2 = b_feat.reshape(1, H)

    embs = []
    for p in range(P):
        g0 = p * part_g
        ii = lax.slice(in_degree, (g0, 0), (g0 + part_g, NUM_FRAG))
        io = lax.slice(out_degree, (g0, 0), (g0 + part_g, NUM_FRAG))
        embs.append(_sc_embedding_sum(
            in_tab, out_tab,
            ii.reshape(NW, per_w // chunk, chunk),
            io.reshape(NW, per_w // chunk, chunk)))

    out = None
    for p in range(P):
        g0 = p * part_g
        frag_p = lax.slice(frag_feature, (g0, 0, 0),
                           (g0 + part_g, NUM_FRAG, H))
        out = _tc_fuse_part(out, frag_p, embs[p], W_feat, b2, graph_token,
                            n_graph, p * (part_g // BLOCK_G))
    return out


# speedup vs baseline: 2.4135x; 2.4135x over previous
import functools

import jax
import jax.numpy as jnp
from jax import lax
from jax.experimental import pallas as pl
from jax.experimental.pallas import tpu as pltpu
from jax.experimental.pallas import tpu_sc as plsc

H = 128
NUM_FRAG = 50
N_CORES = 2
N_SUBCORES = 16
NW = N_CORES * N_SUBCORES
NBUF = 2
P = 4
BLOCK_G = 64


def _sc_embedding_sum(in_tab, out_tab, idx_in3, idx_out3):
    n_chunks, chunk = idx_in3.shape[1], idx_in3.shape[2]
    per_w = n_chunks * chunk
    B = NW * per_w
    mesh = plsc.VectorSubcoreMesh(core_axis_name="c", subcore_axis_name="s")

    @functools.partial(
        pl.kernel,
        mesh=mesh,
        out_type=jax.ShapeDtypeStruct((B, H), jnp.float32),
        scratch_types=[
            pltpu.VMEM_SHARED((512, H), jnp.float32),
            pltpu.VMEM_SHARED((512, H), jnp.float32),
            pltpu.VMEM((n_chunks, chunk), jnp.int32),
            pltpu.VMEM((n_chunks, chunk), jnp.int32),
        ]
        + [pltpu.VMEM((chunk, H), jnp.float32) for _ in range(3 * NBUF)]
        + [pltpu.SemaphoreType.DMA for _ in range(3 * NBUF)],
    )
    def k(in_tab_h, out_tab_h, ii_h, io_h, out_h, in_sp, out_sp, iiB, ioB,
          ri0, ri1, ro0, ro1, o0, o1, si0, si1, so0, so1, sw0, sw1):
        cid = lax.axis_index("c")
        sid = lax.axis_index("s")
        wid = sid * N_CORES + cid
        base = wid * per_w
        ri = [ri0, ri1]
        ro = [ro0, ro1]
        o = [o0, o1]
        si = [si0, si1]
        so = [so0, so1]
        sw = [sw0, sw1]

        @pl.when(sid == 0)
        def _():
            pltpu.sync_copy(in_tab_h, in_sp)
            pltpu.sync_copy(out_tab_h, out_sp)

        pltpu.sync_copy(ii_h.at[wid], iiB)
        pltpu.sync_copy(io_h.at[wid], ioB)
        plsc.subcore_barrier()

        def fill(b, ci):
            pltpu.async_copy(in_sp.at[iiB.at[ci]], ri[b], si[b])
            pltpu.async_copy(out_sp.at[ioB.at[ci]], ro[b], so[b])

        for b in range(NBUF):
            fill(b, b)

        def macro(m, carry):
            for b in range(NBUF):
                ci = NBUF * m + b
                off = base + ci * chunk
                pltpu.make_async_copy(in_sp.at[iiB.at[ci]], ri[b], si[b]).wait()
                pltpu.make_async_copy(out_sp.at[ioB.at[ci]], ro[b], so[b]).wait()

                @pl.when(m > 0)
                def _():
                    pltpu.make_async_copy(
                        o[b], out_h.at[pl.ds(base, chunk)], sw[b]).wait()

                @plsc.parallel_loop(0, chunk, unroll=2)
                def add_row(i):
                    for j in range(H // 16):
                        sl = pl.ds(j * 16, 16)
                        o[b][i, sl] = ri[b][i, sl] + ro[b][i, sl]
                pltpu.async_copy(o[b], out_h.at[pl.ds(off, chunk)], sw[b])

                @pl.when(ci + NBUF < n_chunks)
                def _():
                    fill(b, ci + NBUF)

            return carry

        lax.fori_loop(0, n_chunks // NBUF, macro, 0)

        for b in range(NBUF):
            pltpu.make_async_copy(
                o[b], out_h.at[pl.ds(base, chunk)], sw[b]).wait()

    return k(in_tab, out_tab, idx_in3, idx_out3)


def _tc_body(frag_ref, emb_ref, w_ref, b_ref, tok_ref, out_ref):
    x = frag_ref[...].reshape(NUM_FRAG * BLOCK_G, H)
    feat = lax.dot_general(
        x, w_ref[...], (((1,), (1,)), ((), ())),
        preferred_element_type=jnp.float32,
    )
    feat = feat + emb_ref[...].reshape(NUM_FRAG * BLOCK_G, H) + b_ref[...]
    out_ref[1:, :, :] = feat.reshape(NUM_FRAG, BLOCK_G, H)
    out_ref[0:1, :, :] = jnp.broadcast_to(tok_ref[...][None, :, :],
                                          (1, BLOCK_G, H))


def _tc_fuse_part(prev, frag_t, emb_p3, W, b2, tok, n_graph, off_blocks):
    grid = emb_p3.shape[1] // BLOCK_G

    common = dict(
        grid=(grid,),
        out_specs=pl.BlockSpec((NUM_FRAG + 1, BLOCK_G, H),
                               lambda i: (0, i + off_blocks, 0)),
        out_shape=jax.ShapeDtypeStruct((NUM_FRAG + 1, n_graph, H),
                                       jnp.float32),
        compiler_params=pltpu.CompilerParams(
            dimension_semantics=("arbitrary",)),
    )
    data_specs = [
        pl.BlockSpec((NUM_FRAG, BLOCK_G, H), lambda i: (0, i + off_blocks, 0)),
        pl.BlockSpec((NUM_FRAG, BLOCK_G, H), lambda i: (0, i, 0)),
        pl.BlockSpec((H, H), lambda i: (0, 0)),
        pl.BlockSpec((1, H), lambda i: (0, 0)),
        pl.BlockSpec((1, H), lambda i: (0, 0)),
    ]
    if prev is None:
        return pl.pallas_call(
            _tc_body, in_specs=data_specs, **common,
        )(frag_t, emb_p3, W, b2, tok)

    def body(prev_ref, frag_ref, emb_ref, w_ref, b_ref, tok_ref, out_ref):
        _tc_body(frag_ref, emb_ref, w_ref, b_ref, tok_ref, out_ref)

    return pl.pallas_call(
        body,
        in_specs=[pl.BlockSpec((1, 8, H), lambda i: (0, 0, 0))] + data_specs,
        input_output_aliases={0: 0},
        **common,
    )(prev, frag_t, emb_p3, W, b2, tok)


def kernel(frag_feature, in_degree, out_degree, W_feat, b_feat, in_tab,
           out_tab, graph_token):
    n_graph = frag_feature.shape[0]
    part_g = n_graph // P
    part_b = part_g * NUM_FRAG
    per_w = part_b // NW
    chunk = 128 if per_w % 128 == 0 else 80
    b2 = b_feat.reshape(1, H)

    frag_t = frag_feature.transpose(1, 0, 2)
    ii_t = in_degree.transpose(1, 0)
    io_t = out_degree.transpose(1, 0)

    embs = []
    for p in range(P):
        g0 = p * part_g
        ii = lax.slice(ii_t, (0, g0), (NUM_FRAG, g0 + part_g))
        io = lax.slice(io_t, (0, g0), (NUM_FRAG, g0 + part_g))
        emb = _sc_embedding_sum(
            in_tab, out_tab,
            ii.reshape(NW, per_w // chunk, chunk),
            io.reshape(NW, per_w // chunk, chunk))
        embs.append(emb.reshape(NUM_FRAG, part_g, H))

    out = None
    for p in range(P):
        out = _tc_fuse_part(out, frag_t, embs[p], W_feat, b2, graph_token,
                            n_graph, p * (part_g // BLOCK_G))
    return out.transpose(1, 0, 2)
